# Initial kernel scaffold; baseline (speedup 1.0000x reference)
import functools

import jax
import jax.numpy as jnp
from jax import lax
from jax.experimental import pallas as pl
from jax.experimental.pallas import tpu as pltpu
from jax.experimental.pallas import tpu_sc as plsc

_B = 128
_DIN = 2048
_DHID = 65536
_K = 32
_BH = 2048
_NBLK = _DHID // _BH
_CHUNK = 128
_NCHUNK = _DHID // _CHUNK
_MBH = 16384
_NW = 32
_RPW = _B // _NW


def _encode_body(x_ref, w_ref, be_ref, z_ref):
    z = lax.dot_general(x_ref[...], w_ref[...], (((1,), (1,)), ((), ())),
                        preferred_element_type=jnp.float32)
    z_ref[...] = jnp.maximum(z + be_ref[...], 0.0)


_encode = pl.pallas_call(
    _encode_body,
    grid=(_NBLK,),
    in_specs=[
        pl.BlockSpec((_B, _DIN), lambda i: (0, 0)),
        pl.BlockSpec((_BH, _DIN), lambda i: (i, 0)),
        pl.BlockSpec((1, _BH), lambda i: (0, i)),
    ],
    out_specs=pl.BlockSpec((_B, _BH), lambda i: (0, i)),
    out_shape=jax.ShapeDtypeStruct((_B, _DHID), jnp.float32),
)


def _chunkmax_body(z_ref, m_ref):
    zb = z_ref[...]
    m_ref[...] = jnp.max(zb.reshape(_B, _MBH // _CHUNK, _CHUNK), axis=2)


_chunkmax = pl.pallas_call(
    _chunkmax_body,
    grid=(_DHID // _MBH,),
    in_specs=[pl.BlockSpec((_B, _MBH), lambda i: (0, i))],
    out_specs=pl.BlockSpec((_B, _MBH // _CHUNK), lambda i: (0, i)),
    out_shape=jax.ShapeDtypeStruct((_B, _NCHUNK), jnp.float32),
)


def _select32(read, nv, hist_ref, s_ref, rank):
    lane = lax.iota(jnp.int32, 16)
    ones = jnp.ones((16,), jnp.int32)
    zeros16 = jnp.zeros((16,), jnp.int32)
    prefix = jnp.int32(0)
    r = jnp.int32(rank)
    for level in range(4):
        sh = 24 - 8 * level

        def _zero(j, _):
            hist_ref[pl.ds(j * 16, 16)] = zeros16
            return 0
        lax.fori_loop(0, 256, _zero, 0)

        pfx = prefix

        def _data(i, _):
            v = read(i)
            u = jnp.maximum(plsc.bitcast(v, jnp.int32), 0)
            bin_ = (u >> sh) & 0xFF
            idx = lane * 256 + bin_
            if level == 0:
                plsc.addupdate_scatter(hist_ref, [idx], ones)
            else:
                plsc.addupdate_scatter(hist_ref, [idx], ones,
                                       mask=(u >> (sh + 8)) == pfx)
            return 0
        lax.fori_loop(0, nv, _data, 0)

        s_ref[pl.ds(256, 16)] = zeros16

        def _sfx(jj, carry):
            cnt_vec, csum = carry
            j = 15 - jj
            tot = hist_ref[pl.ds(j * 16, 16)]
            for l in range(1, 16):
                tot = tot + hist_ref[pl.ds(l * 256 + j * 16, 16)]
            sj = jnp.flip(jnp.cumsum(jnp.flip(tot))) + csum
            s_ref[pl.ds(j * 16, 16)] = sj
            cnt = plsc.all_reduce_population_count(sj >= r)
            return (cnt_vec + cnt, csum + jnp.sum(tot))
        cnt_vec, _ = lax.fori_loop(0, 16, _sfx,
                                   (jnp.zeros((16,), jnp.int32), jnp.int32(0)))
        b_sel = jnp.max(cnt_vec) - 1
        r = r - s_ref[b_sel + 1]
        prefix = (prefix << 8) | b_sel
    return plsc.bitcast(jnp.full((16,), prefix, jnp.int32), jnp.float32)


_sc_mesh = plsc.VectorSubcoreMesh(core_axis_name="c", subcore_axis_name="s")


@functools.partial(
    pl.kernel,
    out_type=jax.ShapeDtypeStruct((_B, 16), jnp.float32),
    mesh=_sc_mesh,
    scratch_types=[
        pltpu.VMEM((_NCHUNK,), jnp.float32),
        pltpu.VMEM((544,), jnp.int32),
        pltpu.VMEM((_K, _CHUNK), jnp.float32),
        pltpu.VMEM((16 * 256,), jnp.int32),
        pltpu.VMEM((272,), jnp.int32),
        pltpu.VMEM((16,), jnp.float32),
        pltpu.SemaphoreType.DMA,
    ],
)
def _sc_select(m_hbm, z2_hbm, thr_hbm, m_v, idx_v, cand_v, hist_v, s_v,
               tbuf_v, sem):
    wid = lax.axis_index("s") * 2 + lax.axis_index("c")
    lane = lax.iota(jnp.int32, 16)

    def _row(rr, _):
        row = wid * _RPW + rr
        pltpu.sync_copy(m_hbm.at[row], m_v)
        tb = _select32(lambda i: m_v[pl.ds(i * 16, 16)], _NCHUNK // 16,
                       hist_v, s_v, _K)

        def _coll(j, off):
            v = m_v[pl.ds(j * 16, 16)]
            msk = v >= tb
            gid = row * _NCHUNK + j * 16 + lane
            plsc.store_compressed(idx_v.at[pl.ds(off, 16)], gid, mask=msk)
            return off + jnp.sum(msk.astype(jnp.int32))
        lax.fori_loop(0, _NCHUNK // 16, _coll, jnp.int32(0))

        pltpu.async_copy(z2_hbm.at[idx_v.at[pl.ds(0, _K)]], cand_v, sem).wait()

        def _readc(i):
            return cand_v[i >> 3, pl.ds((i & 7) * 16, 16)]
        tbuf_v[...] = _select32(_readc, _K * _CHUNK // 16, hist_v, s_v, _K)
        pltpu.sync_copy(tbuf_v, thr_hbm.at[row])
        return 0

    lax.fori_loop(0, _RPW, _row, 0)


def _decode_body(z_ref, t_ref, w_ref, bd_ref, o_ref):
    i = pl.program_id(0)
    z = z_ref[...]
    t = t_ref[...][:, 0:1]
    zs = jnp.where(z >= t, z, 0.0)
    part = lax.dot_general(zs, w_ref[...], (((1,), (1,)), ((), ())),
                           preferred_element_type=jnp.float32)

    @pl.when(i == 0)
    def _():
        o_ref[...] = part + bd_ref[...]

    @pl.when(i > 0)
    def _():
        o_ref[...] += part


_decode = pl.pallas_call(
    _decode_body,
    grid=(_NBLK,),
    in_specs=[
        pl.BlockSpec((_B, _BH), lambda i: (0, i)),
        pl.BlockSpec((_B, 16), lambda i: (0, 0)),
        pl.BlockSpec((_DIN, _BH), lambda i: (0, i)),
        pl.BlockSpec((1, _DIN), lambda i: (0, 0)),
    ],
    out_specs=pl.BlockSpec((_B, _DIN), lambda i: (0, 0)),
    out_shape=jax.ShapeDtypeStruct((_B, _DIN), jnp.float32),
)


def kernel(x, W_enc, b_enc, W_dec, b_dec):
    z = _encode(x, W_enc, b_enc.reshape(1, _DHID))
    m = _chunkmax(z)
    thr = _sc_select(m, z.reshape(_B * _NCHUNK, _CHUNK))
    return _decode(z, thr, W_dec, b_dec.reshape(1, _DIN))


# speedup vs baseline: 2.0775x; 2.0775x over previous
import functools

import jax
import jax.numpy as jnp
from jax import lax
from jax.experimental import pallas as pl
from jax.experimental.pallas import tpu as pltpu
from jax.experimental.pallas import tpu_sc as plsc

_B = 128
_DIN = 2048
_DHID = 65536
_K = 32
_BH = 1024
_NBLK = _DHID // _BH
_CHUNK = 128
_NCHUNK = _DHID // _CHUNK
_MBH = 16384
_NW = 32
_RPW = _B // _NW


def _encode_body(x_ref, w_ref, be_ref, z_ref):
    z = lax.dot_general(x_ref[...], w_ref[...], (((1,), (1,)), ((), ())),
                        preferred_element_type=jnp.float32)
    z_ref[...] = jnp.maximum(z + be_ref[...], 0.0)


_encode = pl.pallas_call(
    _encode_body,
    grid=(_NBLK,),
    in_specs=[
        pl.BlockSpec((_B, _DIN), lambda i: (0, 0)),
        pl.BlockSpec((_BH, _DIN), lambda i: (i, 0)),
        pl.BlockSpec((1, _BH), lambda i: (0, i)),
    ],
    out_specs=pl.BlockSpec((_B, _BH), lambda i: (0, i)),
    out_shape=jax.ShapeDtypeStruct((_B, _DHID), jnp.float32),
)


def _chunkmax_body(z_ref, m_ref):
    zb = z_ref[...]
    m_ref[...] = jnp.max(zb.reshape(_B, _MBH // _CHUNK, _CHUNK), axis=2)


_chunkmax = pl.pallas_call(
    _chunkmax_body,
    grid=(_DHID // _MBH,),
    in_specs=[pl.BlockSpec((_B, _MBH), lambda i: (0, i))],
    out_specs=pl.BlockSpec((_B, _MBH // _CHUNK), lambda i: (0, i)),
    out_shape=jax.ShapeDtypeStruct((_B, _NCHUNK), jnp.float32),
)


def _select32(read, nv, hist_ref, s_ref, rank):
    lane = lax.iota(jnp.int32, 16)
    ones = jnp.ones((16,), jnp.int32)
    zeros16 = jnp.zeros((16,), jnp.int32)
    pfx_vec = jnp.zeros((16,), jnp.int32)
    r_vec = jnp.full((16,), rank, jnp.int32)
    for level in range(4):
        sh = 24 - 8 * level

        def _zero(j, _):
            hist_ref[pl.ds(j * 16, 16)] = zeros16
            return 0
        lax.fori_loop(0, 256, _zero, 0)

        pfx = pfx_vec

        def _data(i, _):
            v = read(i)
            u = jnp.maximum(lax.bitcast_convert_type(v, jnp.int32), 0)
            bin_ = (u >> sh) & 0xFF
            idx = lane * 256 + bin_
            if level == 0:
                plsc.addupdate_scatter(hist_ref, [idx], ones)
            else:
                plsc.addupdate_scatter(hist_ref, [idx], ones,
                                       mask=(u >> (sh + 8)) == pfx)
            return 0
        lax.fori_loop(0, nv, _data, 0)

        s_ref[pl.ds(256, 16)] = zeros16

        def _sfx(jj, carry):
            cnt_vec, csum_vec = carry
            j = 15 - jj
            tot = hist_ref[pl.ds(j * 16, 16)]
            for l in range(1, 16):
                tot = tot + hist_ref[pl.ds(l * 256 + j * 16, 16)]
            sj = jnp.flip(jnp.cumsum(jnp.flip(tot))) + csum_vec
            s_ref[pl.ds(j * 16, 16)] = sj
            cnt = plsc.all_reduce_population_count(sj >= r_vec)
            new_csum = plsc.load_gather(s_ref,
                                        [jnp.full((16,), j * 16, jnp.int32)])
            return (cnt_vec + cnt, new_csum)
        cnt_vec, _ = lax.fori_loop(0, 16, _sfx, (zeros16, zeros16))
        b_vec = cnt_vec - 1
        sb1 = plsc.load_gather(s_ref, [b_vec + 1])
        r_vec = r_vec - sb1
        pfx_vec = (pfx_vec << 8) | b_vec
    return pfx_vec


_sc_mesh = plsc.VectorSubcoreMesh(core_axis_name="c", subcore_axis_name="s")


@functools.partial(
    pl.kernel,
    out_type=jax.ShapeDtypeStruct((_B, 16), jnp.int32),
    mesh=_sc_mesh,
    compiler_params=pltpu.CompilerParams(needs_layout_passes=False),
    scratch_types=[
        pltpu.VMEM((_NCHUNK,), jnp.float32),
        pltpu.VMEM((544,), jnp.int32),
        pltpu.VMEM((_K, _CHUNK), jnp.float32),
        pltpu.VMEM((16 * 256,), jnp.int32),
        pltpu.VMEM((272,), jnp.int32),
        pltpu.VMEM((16,), jnp.int32),
        pltpu.SemaphoreType.DMA,
    ],
)
def _sc_select(m_hbm, z2_hbm, thr_hbm, m_v, idx_v, cand_v, hist_v, s_v,
               tbuf_v, sem):
    wid = lax.axis_index("s") * 2 + lax.axis_index("c")
    lane = lax.iota(jnp.int32, 16)

    def _row(rr, _):
        row = wid * _RPW + rr
        pltpu.sync_copy(m_hbm.at[row], m_v)
        tb = _select32(lambda i: m_v[pl.ds(i * 16, 16)], _NCHUNK // 16,
                       hist_v, s_v, _K)

        def _coll_strict(j, off):
            v = m_v[pl.ds(j * 16, 16)]
            u = jnp.maximum(lax.bitcast_convert_type(v, jnp.int32), 0)
            msk = u > tb
            gid = row * _NCHUNK + j * 16 + lane
            plsc.store_compressed(idx_v.at[pl.ds(off, 16)], gid, mask=msk)
            return off + jnp.sum(msk.astype(jnp.int32))
        off1 = lax.fori_loop(0, _NCHUNK // 16, _coll_strict, jnp.int32(0))

        def _coll_ties(j, off):
            v = m_v[pl.ds(j * 16, 16)]
            u = jnp.maximum(lax.bitcast_convert_type(v, jnp.int32), 0)
            msk = u == tb
            gid = row * _NCHUNK + j * 16 + lane
            plsc.store_compressed(idx_v.at[pl.ds(off, 16)], gid, mask=msk)
            return off + jnp.sum(msk.astype(jnp.int32))
        lax.fori_loop(0, _NCHUNK // 16, _coll_ties, off1)

        pltpu.async_copy(z2_hbm.at[idx_v.at[pl.ds(0, _K)]], cand_v, sem).wait()

        def _readc(i):
            return cand_v[i >> 3, pl.ds((i & 7) * 16, 16)]
        tbuf_v[...] = _select32(_readc, _K * _CHUNK // 16, hist_v, s_v, _K)
        pltpu.sync_copy(tbuf_v, thr_hbm.at[row])
        return 0

    lax.fori_loop(0, _RPW, _row, 0)


def _decode_body(z_ref, t_ref, w_ref, bd_ref, o_ref):
    i = pl.program_id(0)
    z = z_ref[...]
    t = t_ref[...][:, 0:1]
    zs = jnp.where(z >= t, z, 0.0)
    part = lax.dot_general(zs, w_ref[...], (((1,), (1,)), ((), ())),
                           preferred_element_type=jnp.float32)

    @pl.when(i == 0)
    def _():
        o_ref[...] = part + bd_ref[...]

    @pl.when(i > 0)
    def _():
        o_ref[...] += part


_decode = pl.pallas_call(
    _decode_body,
    grid=(_NBLK,),
    in_specs=[
        pl.BlockSpec((_B, _BH), lambda i: (0, i)),
        pl.BlockSpec((_B, 16), lambda i: (0, 0)),
        pl.BlockSpec((_DIN, _BH), lambda i: (0, i)),
        pl.BlockSpec((1, _DIN), lambda i: (0, 0)),
    ],
    out_specs=pl.BlockSpec((_B, _DIN), lambda i: (0, 0)),
    out_shape=jax.ShapeDtypeStruct((_B, _DIN), jnp.float32),
)


def kernel(x, W_enc, b_enc, W_dec, b_dec):
    z = _encode(x, W_enc, b_enc.reshape(1, _DHID))
    m = _chunkmax(z)
    thr_bits = _sc_select(m, z.reshape(_B * _NCHUNK, _CHUNK))
    thr = lax.bitcast_convert_type(thr_bits, jnp.float32)
    return _decode(z, thr, W_dec, b_dec.reshape(1, _DIN))


# speedup vs baseline: 2.2153x; 1.0663x over previous
import functools

import jax
import jax.numpy as jnp
from jax import lax
from jax.experimental import pallas as pl
from jax.experimental.pallas import tpu as pltpu
from jax.experimental.pallas import tpu_sc as plsc

_B = 128
_DIN = 2048
_DHID = 65536
_K = 32
_BH = 1024
_NBLK = _DHID // _BH
_CHUNK = 128
_NCHUNK = _DHID // _CHUNK
_MBH = 16384
_NW = 32
_RPW = _B // _NW


def _encode_body(x_ref, w_ref, be_ref, z_ref):
    z = lax.dot_general(x_ref[...], w_ref[...], (((1,), (1,)), ((), ())),
                        preferred_element_type=jnp.float32)
    z_ref[...] = jnp.maximum(z + be_ref[...], 0.0)


_encode = pl.pallas_call(
    _encode_body,
    grid=(_NBLK,),
    in_specs=[
        pl.BlockSpec((_B, _DIN), lambda i: (0, 0)),
        pl.BlockSpec((_BH, _DIN), lambda i: (i, 0)),
        pl.BlockSpec((1, _BH), lambda i: (0, i)),
    ],
    out_specs=pl.BlockSpec((_B, _BH), lambda i: (0, i)),
    out_shape=jax.ShapeDtypeStruct((_B, _DHID), jnp.float32),
)


def _chunkmax_body(z_ref, m_ref):
    zb = z_ref[...]
    m_ref[...] = jnp.max(zb.reshape(_B, _MBH // _CHUNK, _CHUNK), axis=2)


_chunkmax = pl.pallas_call(
    _chunkmax_body,
    grid=(_DHID // _MBH,),
    in_specs=[pl.BlockSpec((_B, _MBH), lambda i: (0, i))],
    out_specs=pl.BlockSpec((_B, _MBH // _CHUNK), lambda i: (0, i)),
    out_shape=jax.ShapeDtypeStruct((_B, _NCHUNK), jnp.float32),
)


def _select32(read, nv, hist_ref, s_ref, rank):
    lane = lax.iota(jnp.int32, 16)
    ones = jnp.ones((16,), jnp.int32)
    zeros16 = jnp.zeros((16,), jnp.int32)
    pfx_vec = jnp.zeros((16,), jnp.int32)
    r_vec = jnp.full((16,), rank, jnp.int32)
    for level in range(4):
        sh = 24 - 8 * level

        def _zero(j, _):
            hist_ref[pl.ds(j * 16, 16)] = zeros16
            return 0
        lax.fori_loop(0, 256, _zero, 0, unroll=8)

        pfx = pfx_vec

        def _data(i, _):
            v = read(i)
            u = jnp.maximum(lax.bitcast_convert_type(v, jnp.int32), 0)
            bin_ = (u >> sh) & 0xFF
            idx = lane * 256 + bin_
            if level == 0:
                plsc.addupdate_scatter(hist_ref, [idx], ones)
            else:
                plsc.addupdate_scatter(hist_ref, [idx], ones,
                                       mask=(u >> (sh + 8)) == pfx)
            return 0
        lax.fori_loop(0, nv, _data, 0, unroll=8)

        s_ref[pl.ds(256, 16)] = zeros16

        def _sfx(jj, carry):
            cnt_vec, csum_vec = carry
            j = 15 - jj
            tot = hist_ref[pl.ds(j * 16, 16)]
            for l in range(1, 16):
                tot = tot + hist_ref[pl.ds(l * 256 + j * 16, 16)]
            sj = jnp.flip(jnp.cumsum(jnp.flip(tot))) + csum_vec
            s_ref[pl.ds(j * 16, 16)] = sj
            cnt = plsc.all_reduce_population_count(sj >= r_vec)
            new_csum = plsc.load_gather(s_ref,
                                        [jnp.full((16,), j * 16, jnp.int32)])
            return (cnt_vec + cnt, new_csum)
        cnt_vec, _ = lax.fori_loop(0, 16, _sfx, (zeros16, zeros16), unroll=2)
        b_vec = cnt_vec - 1
        sb1 = plsc.load_gather(s_ref, [b_vec + 1])
        r_vec = r_vec - sb1
        pfx_vec = (pfx_vec << 8) | b_vec
    return pfx_vec


_sc_mesh = plsc.VectorSubcoreMesh(core_axis_name="c", subcore_axis_name="s")


@functools.partial(
    pl.kernel,
    out_type=jax.ShapeDtypeStruct((_B, 16), jnp.int32),
    mesh=_sc_mesh,
    compiler_params=pltpu.CompilerParams(needs_layout_passes=False),
    scratch_types=[
        pltpu.VMEM((_NCHUNK,), jnp.float32),
        pltpu.VMEM((544,), jnp.int32),
        pltpu.VMEM((_K, _CHUNK), jnp.float32),
        pltpu.VMEM((16 * 256,), jnp.int32),
        pltpu.VMEM((272,), jnp.int32),
        pltpu.VMEM((16,), jnp.int32),
        pltpu.SemaphoreType.DMA,
    ],
)
def _sc_select(m_hbm, z2_hbm, thr_hbm, m_v, idx_v, cand_v, hist_v, s_v,
               tbuf_v, sem):
    wid = lax.axis_index("s") * 2 + lax.axis_index("c")
    lane = lax.iota(jnp.int32, 16)

    def _row(rr, _):
        row = wid * _RPW + rr
        pltpu.sync_copy(m_hbm.at[row], m_v)
        tb = _select32(lambda i: m_v[pl.ds(i * 16, 16)], _NCHUNK // 16,
                       hist_v, s_v, _K)

        def _coll_strict(j, off):
            v = m_v[pl.ds(j * 16, 16)]
            u = jnp.maximum(lax.bitcast_convert_type(v, jnp.int32), 0)
            msk = u > tb
            gid = row * _NCHUNK + j * 16 + lane
            plsc.store_compressed(idx_v.at[pl.ds(off, 16)], gid, mask=msk)
            return off + jnp.sum(msk.astype(jnp.int32))
        off1 = lax.fori_loop(0, _NCHUNK // 16, _coll_strict, jnp.int32(0), unroll=4)

        def _coll_ties(j, off):
            v = m_v[pl.ds(j * 16, 16)]
            u = jnp.maximum(lax.bitcast_convert_type(v, jnp.int32), 0)
            msk = u == tb
            gid = row * _NCHUNK + j * 16 + lane
            plsc.store_compressed(idx_v.at[pl.ds(off, 16)], gid, mask=msk)
            return off + jnp.sum(msk.astype(jnp.int32))
        lax.fori_loop(0, _NCHUNK // 16, _coll_ties, off1, unroll=4)

        pltpu.async_copy(z2_hbm.at[idx_v.at[pl.ds(0, _K)]], cand_v, sem).wait()

        def _readc(i):
            return cand_v[i >> 3, pl.ds((i & 7) * 16, 16)]
        tbuf_v[...] = _select32(_readc, _K * _CHUNK // 16, hist_v, s_v, _K)
        pltpu.sync_copy(tbuf_v, thr_hbm.at[row])
        return 0

    lax.fori_loop(0, _RPW, _row, 0)


def _decode_body(z_ref, t_ref, w_ref, bd_ref, o_ref):
    i = pl.program_id(0)
    z = z_ref[...]
    t = t_ref[...][:, 0:1]
    zs = jnp.where(z >= t, z, 0.0)
    part = lax.dot_general(zs, w_ref[...], (((1,), (1,)), ((), ())),
                           preferred_element_type=jnp.float32)

    @pl.when(i == 0)
    def _():
        o_ref[...] = part + bd_ref[...]

    @pl.when(i > 0)
    def _():
        o_ref[...] += part


_decode = pl.pallas_call(
    _decode_body,
    grid=(_NBLK,),
    in_specs=[
        pl.BlockSpec((_B, _BH), lambda i: (0, i)),
        pl.BlockSpec((_B, 16), lambda i: (0, 0)),
        pl.BlockSpec((_DIN, _BH), lambda i: (0, i)),
        pl.BlockSpec((1, _DIN), lambda i: (0, 0)),
    ],
    out_specs=pl.BlockSpec((_B, _DIN), lambda i: (0, 0)),
    out_shape=jax.ShapeDtypeStruct((_B, _DIN), jnp.float32),
)


def kernel(x, W_enc, b_enc, W_dec, b_dec):
    z = _encode(x, W_enc, b_enc.reshape(1, _DHID))
    m = _chunkmax(z)
    thr_bits = _sc_select(m, z.reshape(_B * _NCHUNK, _CHUNK))
    thr = lax.bitcast_convert_type(thr_bits, jnp.float32)
    return _decode(z, thr, W_dec, b_dec.reshape(1, _DIN))
